# baseline (speedup 1.0000x reference)
import functools

import jax
import jax.numpy as jnp
from jax import lax
from jax.experimental import pallas as pl
from jax.experimental.pallas import tpu as pltpu
from jax.experimental.pallas import tpu_sc as plsc

N = 10000
NP = 10240
E = 320000
D = 128
NC = 2
NS = 16
NW = NC * NS
EPW = 10240
EPAD = NW * EPW
CH = 80
NCH = EPW // CH
IB = 8
NG = NCH // IB
RPT = NP // NS
ZR = 128

f32 = jnp.float32


def _sc_agg(h, z, src3, dst3):
    mesh = plsc.VectorSubcoreMesh(core_axis_name="c", subcore_axis_name="s",
                                  num_cores=NC, num_subcores=NS)

    @functools.partial(
        pl.kernel,
        out_type=jax.ShapeDtypeStruct((NC, NP, D), f32),
        mesh=mesh,
        scratch_types=[
            pltpu.VMEM((2, IB, CH), jnp.int32),
            pltpu.VMEM((2, IB, CH), jnp.int32),
            pltpu.VMEM((4 * CH, D), f32),
            pltpu.VMEM_SHARED((NP, D), f32),
            pltpu.SemaphoreType.DMA,
            pltpu.SemaphoreType.DMA,
            pltpu.SemaphoreType.DMA,
            pltpu.SemaphoreType.DMA,
            pltpu.SemaphoreType.DMA,
        ],
    )
    def run(h_hbm, z_hbm, src_hbm, dst_hbm, sums_hbm, src_v, dst_v, rows_v,
            acc, sem_g, sem_s0, sem_s1, sem_s2, sem_s3):
        bufs = tuple(rows_v.at[pl.ds(b * CH, CH)] for b in range(4))
        sem_s = (sem_s0, sem_s1, sem_s2, sem_s3)
        cid = lax.axis_index("c")
        sid = lax.axis_index("s")
        wid = cid * NS + sid
        base = sid * RPT

        pltpu.sync_copy(z_hbm.at[pl.ds(base, RPT)], acc.at[pl.ds(base, RPT)])
        plsc.subcore_barrier()

        def drain_scatter(par):
            pltpu.make_async_copy(
                z_hbm.at[pl.ds(0, CH)],
                rows_v.at[pl.ds(par * CH, CH)],
                sem_s[par]).wait()

        def drain_gather(par):
            pltpu.make_async_copy(
                z_hbm.at[pl.ds(0, CH)],
                rows_v.at[pl.ds(par * CH, CH)],
                sem_g).wait()

        pltpu.sync_copy(src_hbm.at[wid, pl.ds(0, IB)], src_v.at[0])
        pltpu.sync_copy(dst_hbm.at[wid, pl.ds(0, IB)], dst_v.at[0])
        pltpu.async_copy(h_hbm.at[src_v.at[0, 0]], bufs[0], sem_g)

        def group(g, carry):
            rp = lax.rem(g, 2)
            rn = lax.rem(g + 1, 2)

            @pl.when(g + 1 < NG)
            def _():
                pltpu.sync_copy(src_hbm.at[wid, pl.ds((g + 1) * IB, IB)],
                                src_v.at[rn])
                pltpu.sync_copy(dst_hbm.at[wid, pl.ds((g + 1) * IB, IB)],
                                dst_v.at[rn])
            for j in range(IB):
                par = j % 4
                nxt = (j + 1) % 4
                t = g * IB + j

                @pl.when(t >= 3)
                def _():
                    drain_scatter(nxt)

                @pl.when(t + 1 < NCH)
                def _():
                    if j == IB - 1:
                        pltpu.async_copy(h_hbm.at[src_v.at[rn, 0]],
                                         bufs[nxt], sem_g)
                    else:
                        pltpu.async_copy(h_hbm.at[src_v.at[rp, j + 1]],
                                         bufs[nxt], sem_g)

                drain_gather(par)
                pltpu.async_copy(bufs[par], acc.at[dst_v.at[rp, j]],
                                 sem_s[par], add=True)
            return carry
        lax.fori_loop(0, NG, group, 0)

        for c in range(NCH - 3, NCH):
            drain_scatter(c % 4)

        plsc.subcore_barrier()

        for k in range(RPT // ZR):
            r0 = base + k * ZR
            pltpu.sync_copy(acc.at[pl.ds(r0, ZR)], sums_hbm.at[cid, pl.ds(r0, ZR)])

    return run(h, z, src3, dst3)


def _sc_cnt(z, dst3):
    mesh = plsc.VectorSubcoreMesh(core_axis_name="c", subcore_axis_name="s",
                                  num_cores=NC, num_subcores=NS)

    @functools.partial(
        pl.kernel,
        out_type=jax.ShapeDtypeStruct((NC, NP, D), f32),
        mesh=mesh,
        scratch_types=[
            pltpu.VMEM((IB, CH), jnp.int32),
            pltpu.VMEM((CH, D), f32),
            pltpu.VMEM_SHARED((NP, D), f32),
        ],
    )
    def run(z_hbm, dst_hbm, cnts_hbm, dst_v, ones_v, c_acc):
        cid = lax.axis_index("c")
        sid = lax.axis_index("s")
        wid = cid * NS + sid
        base = sid * RPT

        def orow(i, carry):
            for k in range(D // 16):
                ones_v[i, pl.ds(k * 16, 16)] = jnp.ones((16,), f32)
            return carry
        lax.fori_loop(0, CH, orow, 0)

        pltpu.sync_copy(z_hbm.at[pl.ds(base, RPT)], c_acc.at[pl.ds(base, RPT)])
        plsc.subcore_barrier()

        def group(g, carry):
            pltpu.sync_copy(dst_hbm.at[wid, pl.ds(g * IB, IB)], dst_v)

            def chunk(j, carry2):
                pltpu.sync_copy(ones_v, c_acc.at[dst_v.at[j]], add=True)
                return carry2
            return lax.fori_loop(0, IB, chunk, carry)
        lax.fori_loop(0, NG, group, 0)

        plsc.subcore_barrier()
        for k in range(RPT // ZR):
            r0 = base + k * ZR
            pltpu.sync_copy(c_acc.at[pl.ds(r0, ZR)], cnts_hbm.at[cid, pl.ds(r0, ZR)])

    return run(z, dst3)


R = 1024
GRID = NP // R

_ROWB = pl.BlockSpec((R, D), lambda i: (i, 0))
_FULLW = pl.BlockSpec((D, D), lambda i: (0, 0))
_ROWV = pl.BlockSpec((1, D), lambda i: (0, 0))


def _proj_body(x_ref, w_ref, b_ref, o_ref):
    o_ref[...] = jnp.maximum(
        jnp.dot(x_ref[...], w_ref[...], preferred_element_type=f32) + b_ref[...],
        0.0)


def _proj(x, W, b):
    return pl.pallas_call(
        _proj_body,
        grid=(GRID,),
        in_specs=[_ROWB, _FULLW, _ROWV],
        out_specs=_ROWB,
        out_shape=jax.ShapeDtypeStruct((NP, D), f32),
    )(x, W, b.reshape(1, D))


def _sage_compute(s_ref, c_ref, h_ref, wl_ref, bl_ref, wr_ref, g_ref, be_ref):
    s = s_ref[0] + s_ref[1]
    cnt = (c_ref[0] + c_ref[1])[:, 0:1]
    agg = s / jnp.maximum(cnt, 1.0)
    out = (jnp.dot(agg, wl_ref[...], preferred_element_type=f32) + bl_ref[...]
           + jnp.dot(h_ref[...], wr_ref[...], preferred_element_type=f32))
    mu = jnp.mean(out, axis=1, keepdims=True)
    var = jnp.mean((out - mu) ** 2, axis=1, keepdims=True)
    out = (out - mu) / jnp.sqrt(var + 1e-5) * g_ref[...] + be_ref[...]
    return jnp.maximum(out, 0.0)


def _sage_body(s_ref, c_ref, h_ref, wl_ref, bl_ref, wr_ref, g_ref, be_ref,
               o_ref):
    o_ref[...] = _sage_compute(s_ref, c_ref, h_ref, wl_ref, bl_ref, wr_ref,
                               g_ref, be_ref)


_SAGE_SPECS = [pl.BlockSpec((NC, R, D), lambda i: (0, i, 0)),
               pl.BlockSpec((NC, R, D), lambda i: (0, i, 0)),
               _ROWB, _FULLW, _ROWV, _FULLW, _ROWV, _ROWV]


def _sage(sums, cnts, h, W_l, b_l, W_r, g, be):
    return pl.pallas_call(
        _sage_body,
        grid=(GRID,),
        in_specs=_SAGE_SPECS,
        out_specs=_ROWB,
        out_shape=jax.ShapeDtypeStruct((NP, D), f32),
    )(sums, cnts, h, W_l, b_l.reshape(1, D), W_r, g.reshape(1, D),
      be.reshape(1, D))


def _head_body(s_ref, c_ref, h_ref, wl_ref, bl_ref, wr_ref, g_ref, be_ref,
               h0_ref, w1_ref, b1_ref, w2_ref, b2_ref, o_ref):
    h2 = _sage_compute(s_ref, c_ref, h_ref, wl_ref, bl_ref, wr_ref, g_ref,
                       be_ref)
    hr = h2 + h0_ref[...]
    hid = jnp.maximum(
        jnp.dot(hr, w1_ref[...], preferred_element_type=f32) + b1_ref[...], 0.0)
    o_ref[...] = jnp.dot(hid, w2_ref[...], preferred_element_type=f32) + b2_ref[...]


def _head(sums, cnts, h, W_l, b_l, W_r, g, be, h0, W1, b1, W2, b2):
    HH = D // 2
    return pl.pallas_call(
        _head_body,
        grid=(GRID,),
        in_specs=_SAGE_SPECS + [
            _ROWB,
            pl.BlockSpec((D, HH), lambda i: (0, 0)),
            pl.BlockSpec((1, HH), lambda i: (0, 0)),
            pl.BlockSpec((HH, 1), lambda i: (0, 0)),
            pl.BlockSpec((1, 1), lambda i: (0, 0))],
        out_specs=pl.BlockSpec((R, 1), lambda i: (i, 0)),
        out_shape=jax.ShapeDtypeStruct((NP, 1), f32),
    )(sums, cnts, h, W_l, b_l.reshape(1, D), W_r, g.reshape(1, D),
      be.reshape(1, D), h0, W1, b1.reshape(1, HH), W2, b2.reshape(1, 1))


def kernel(x, edge_index, W_in, b_in, W_l0, b_l0, W_r0, g0, be0,
           W_l1, b_l1, W_r1, g1, be1, W_out1, b_out1, W_out2, b_out2):
    ei = edge_index.astype(jnp.int32)
    pad_n = EPAD - E
    src3 = jnp.concatenate(
        [ei[0], jnp.zeros((pad_n,), jnp.int32)]).reshape(NW, NCH, CH)
    dst3 = jnp.concatenate(
        [ei[1], jnp.full((pad_n,), NP - 1, jnp.int32)]).reshape(NW, NCH, CH)

    x_pad = jnp.pad(x, ((0, NP - N), (0, 0)))
    z = jnp.zeros((NP, D), f32)
    h0 = _proj(x_pad, W_in, b_in)
    cnts = _sc_cnt(z, dst3)
    sums0 = _sc_agg(h0, z, src3, dst3)
    h1 = _sage(sums0, cnts, h0, W_l0, b_l0, W_r0, g0, be0)
    sums1 = _sc_agg(h1, z, src3, dst3)
    pred = _head(sums1, cnts, h1, W_l1, b_l1, W_r1, g1, be1,
                 h0, W_out1, b_out1, W_out2, b_out2)
    return pred[:N]


# speedup vs baseline: 1.0285x; 1.0285x over previous
import functools

import jax
import jax.numpy as jnp
from jax import lax
from jax.experimental import pallas as pl
from jax.experimental.pallas import tpu as pltpu
from jax.experimental.pallas import tpu_sc as plsc

N = 10000
NP = 10240
E = 320000
D = 128
NC = 2
NS = 16
NW = NC * NS
EPW = 10240
EPAD = NW * EPW
CH = 64
NCH = EPW // CH
IB = 8
NG = NCH // IB
RPT = NP // NS
ZR = 128

f32 = jnp.float32


def _sc_agg(h, z, src3, dst3):
    mesh = plsc.VectorSubcoreMesh(core_axis_name="c", subcore_axis_name="s",
                                  num_cores=NC, num_subcores=NS)

    @functools.partial(
        pl.kernel,
        out_type=jax.ShapeDtypeStruct((NC, NP, D), f32),
        mesh=mesh,
        scratch_types=[
            pltpu.VMEM((2, IB, CH), jnp.int32),
            pltpu.VMEM((2, IB, CH), jnp.int32),
            pltpu.VMEM((4 * CH, D), f32),
            pltpu.VMEM_SHARED((NP, D), f32),
            pltpu.SemaphoreType.DMA,
            pltpu.SemaphoreType.DMA,
            pltpu.SemaphoreType.DMA,
            pltpu.SemaphoreType.DMA,
            pltpu.SemaphoreType.DMA,
        ],
    )
    def run(h_hbm, z_hbm, src_hbm, dst_hbm, sums_hbm, src_v, dst_v, rows_v,
            acc, sem_g, sem_s0, sem_s1, sem_s2, sem_s3):
        bufs = tuple(rows_v.at[pl.ds(b * CH, CH)] for b in range(4))
        sem_s = (sem_s0, sem_s1, sem_s2, sem_s3)
        cid = lax.axis_index("c")
        sid = lax.axis_index("s")
        wid = cid * NS + sid
        base = sid * RPT

        pltpu.sync_copy(z_hbm.at[pl.ds(base, RPT)], acc.at[pl.ds(base, RPT)])
        plsc.subcore_barrier()

        def drain_scatter(par):
            pltpu.make_async_copy(
                z_hbm.at[pl.ds(0, CH)],
                rows_v.at[pl.ds(par * CH, CH)],
                sem_s[par]).wait()

        def drain_gather(par):
            pltpu.make_async_copy(
                z_hbm.at[pl.ds(0, CH)],
                rows_v.at[pl.ds(par * CH, CH)],
                sem_g).wait()

        pltpu.sync_copy(src_hbm.at[wid, pl.ds(0, IB)], src_v.at[0])
        pltpu.sync_copy(dst_hbm.at[wid, pl.ds(0, IB)], dst_v.at[0])
        pltpu.async_copy(h_hbm.at[src_v.at[0, 0]], bufs[0], sem_g)

        def group(g, carry):
            rp = lax.rem(g, 2)
            rn = lax.rem(g + 1, 2)

            @pl.when(g + 1 < NG)
            def _():
                pltpu.sync_copy(src_hbm.at[wid, pl.ds((g + 1) * IB, IB)],
                                src_v.at[rn])
                pltpu.sync_copy(dst_hbm.at[wid, pl.ds((g + 1) * IB, IB)],
                                dst_v.at[rn])
            for j in range(IB):
                par = j % 4
                nxt = (j + 1) % 4
                t = g * IB + j

                @pl.when(t >= 3)
                def _():
                    drain_scatter(nxt)

                @pl.when(t + 1 < NCH)
                def _():
                    if j == IB - 1:
                        pltpu.async_copy(h_hbm.at[src_v.at[rn, 0]],
                                         bufs[nxt], sem_g)
                    else:
                        pltpu.async_copy(h_hbm.at[src_v.at[rp, j + 1]],
                                         bufs[nxt], sem_g)

                drain_gather(par)
                pltpu.async_copy(bufs[par], acc.at[dst_v.at[rp, j]],
                                 sem_s[par], add=True)
            return carry
        lax.fori_loop(0, NG, group, 0)

        for c in range(NCH - 3, NCH):
            drain_scatter(c % 4)

        plsc.subcore_barrier()

        for k in range(RPT // ZR):
            r0 = base + k * ZR
            pltpu.sync_copy(acc.at[pl.ds(r0, ZR)], sums_hbm.at[cid, pl.ds(r0, ZR)])

    return run(h, z, src3, dst3)


def _sc_cnt(z, dst3):
    mesh = plsc.VectorSubcoreMesh(core_axis_name="c", subcore_axis_name="s",
                                  num_cores=NC, num_subcores=NS)

    @functools.partial(
        pl.kernel,
        out_type=jax.ShapeDtypeStruct((NC, NP, D), f32),
        mesh=mesh,
        scratch_types=[
            pltpu.VMEM((2, IB, CH), jnp.int32),
            pltpu.VMEM((CH, D), f32),
            pltpu.VMEM_SHARED((NP, D), f32),
            pltpu.SemaphoreType.DMA,
        ],
    )
    def run(z_hbm, dst_hbm, cnts_hbm, dst_v, ones_v, c_acc, sem_s):
        cid = lax.axis_index("c")
        sid = lax.axis_index("s")
        wid = cid * NS + sid
        base = sid * RPT

        def orow(i, carry):
            for k in range(D // 16):
                ones_v[i, pl.ds(k * 16, 16)] = jnp.ones((16,), f32)
            return carry
        lax.fori_loop(0, CH, orow, 0)

        pltpu.sync_copy(z_hbm.at[pl.ds(base, RPT)], c_acc.at[pl.ds(base, RPT)])
        plsc.subcore_barrier()

        pltpu.sync_copy(dst_hbm.at[wid, pl.ds(0, IB)], dst_v.at[0])

        def group(g, carry):
            rp = lax.rem(g, 2)
            rn = lax.rem(g + 1, 2)

            @pl.when(g >= 1)
            def _():
                for _j in range(IB):
                    pltpu.make_async_copy(z_hbm.at[pl.ds(0, CH)], ones_v,
                                          sem_s).wait()

            @pl.when(g + 1 < NG)
            def _():
                pltpu.sync_copy(dst_hbm.at[wid, pl.ds((g + 1) * IB, IB)],
                                dst_v.at[rn])
            for j in range(IB):
                pltpu.async_copy(ones_v, c_acc.at[dst_v.at[rp, j]], sem_s,
                                 add=True)
            return carry
        lax.fori_loop(0, NG, group, 0)
        for _j in range(IB):
            pltpu.make_async_copy(z_hbm.at[pl.ds(0, CH)], ones_v,
                                  sem_s).wait()

        plsc.subcore_barrier()
        for k in range(RPT // ZR):
            r0 = base + k * ZR
            pltpu.sync_copy(c_acc.at[pl.ds(r0, ZR)], cnts_hbm.at[cid, pl.ds(r0, ZR)])

    return run(z, dst3)


R = 1024
GRID = NP // R

_ROWB = pl.BlockSpec((R, D), lambda i: (i, 0))
_FULLW = pl.BlockSpec((D, D), lambda i: (0, 0))
_ROWV = pl.BlockSpec((1, D), lambda i: (0, 0))


def _proj_body(x_ref, w_ref, b_ref, o_ref):
    o_ref[...] = jnp.maximum(
        jnp.dot(x_ref[...], w_ref[...], preferred_element_type=f32) + b_ref[...],
        0.0)


def _proj(x, W, b):
    return pl.pallas_call(
        _proj_body,
        grid=(GRID,),
        in_specs=[_ROWB, _FULLW, _ROWV],
        out_specs=_ROWB,
        out_shape=jax.ShapeDtypeStruct((NP, D), f32),
    )(x, W, b.reshape(1, D))


def _sage_compute(s_ref, c_ref, h_ref, wl_ref, bl_ref, wr_ref, g_ref, be_ref):
    s = s_ref[0] + s_ref[1]
    cnt = (c_ref[0] + c_ref[1])[:, 0:1]
    agg = s / jnp.maximum(cnt, 1.0)
    out = (jnp.dot(agg, wl_ref[...], preferred_element_type=f32) + bl_ref[...]
           + jnp.dot(h_ref[...], wr_ref[...], preferred_element_type=f32))
    mu = jnp.mean(out, axis=1, keepdims=True)
    var = jnp.mean((out - mu) ** 2, axis=1, keepdims=True)
    out = (out - mu) / jnp.sqrt(var + 1e-5) * g_ref[...] + be_ref[...]
    return jnp.maximum(out, 0.0)


def _sage_body(s_ref, c_ref, h_ref, wl_ref, bl_ref, wr_ref, g_ref, be_ref,
               o_ref):
    o_ref[...] = _sage_compute(s_ref, c_ref, h_ref, wl_ref, bl_ref, wr_ref,
                               g_ref, be_ref)


_SAGE_SPECS = [pl.BlockSpec((NC, R, D), lambda i: (0, i, 0)),
               pl.BlockSpec((NC, R, D), lambda i: (0, i, 0)),
               _ROWB, _FULLW, _ROWV, _FULLW, _ROWV, _ROWV]


def _sage(sums, cnts, h, W_l, b_l, W_r, g, be):
    return pl.pallas_call(
        _sage_body,
        grid=(GRID,),
        in_specs=_SAGE_SPECS,
        out_specs=_ROWB,
        out_shape=jax.ShapeDtypeStruct((NP, D), f32),
    )(sums, cnts, h, W_l, b_l.reshape(1, D), W_r, g.reshape(1, D),
      be.reshape(1, D))


def _head_body(s_ref, c_ref, h_ref, wl_ref, bl_ref, wr_ref, g_ref, be_ref,
               h0_ref, w1_ref, b1_ref, w2_ref, b2_ref, o_ref):
    h2 = _sage_compute(s_ref, c_ref, h_ref, wl_ref, bl_ref, wr_ref, g_ref,
                       be_ref)
    hr = h2 + h0_ref[...]
    hid = jnp.maximum(
        jnp.dot(hr, w1_ref[...], preferred_element_type=f32) + b1_ref[...], 0.0)
    o_ref[...] = jnp.dot(hid, w2_ref[...], preferred_element_type=f32) + b2_ref[...]


def _head(sums, cnts, h, W_l, b_l, W_r, g, be, h0, W1, b1, W2, b2):
    HH = D // 2
    return pl.pallas_call(
        _head_body,
        grid=(GRID,),
        in_specs=_SAGE_SPECS + [
            _ROWB,
            pl.BlockSpec((D, HH), lambda i: (0, 0)),
            pl.BlockSpec((1, HH), lambda i: (0, 0)),
            pl.BlockSpec((HH, 1), lambda i: (0, 0)),
            pl.BlockSpec((1, 1), lambda i: (0, 0))],
        out_specs=pl.BlockSpec((R, 1), lambda i: (i, 0)),
        out_shape=jax.ShapeDtypeStruct((NP, 1), f32),
    )(sums, cnts, h, W_l, b_l.reshape(1, D), W_r, g.reshape(1, D),
      be.reshape(1, D), h0, W1, b1.reshape(1, HH), W2, b2.reshape(1, 1))


def kernel(x, edge_index, W_in, b_in, W_l0, b_l0, W_r0, g0, be0,
           W_l1, b_l1, W_r1, g1, be1, W_out1, b_out1, W_out2, b_out2):
    ei = edge_index.astype(jnp.int32)
    pad_n = EPAD - E
    src3 = jnp.concatenate(
        [ei[0], jnp.zeros((pad_n,), jnp.int32)]).reshape(NW, NCH, CH)
    dst3 = jnp.concatenate(
        [ei[1], jnp.full((pad_n,), NP - 1, jnp.int32)]).reshape(NW, NCH, CH)

    x_pad = jnp.pad(x, ((0, NP - N), (0, 0)))
    z = jnp.zeros((NP, D), f32)
    h0 = _proj(x_pad, W_in, b_in)
    cnts = _sc_cnt(z, dst3)
    sums0 = _sc_agg(h0, z, src3, dst3)
    h1 = _sage(sums0, cnts, h0, W_l0, b_l0, W_r0, g0, be0)
    sums1 = _sc_agg(h1, z, src3, dst3)
    pred = _head(sums1, cnts, h1, W_l1, b_l1, W_r1, g1, be1,
                 h0, W_out1, b_out1, W_out2, b_out2)
    return pred[:N]
